# Initial kernel scaffold; baseline (speedup 1.0000x reference)
import jax
import jax.numpy as jnp
from jax.experimental import pallas as pl


def kernel(x, context, attn_indices, bias, Wq, Wkv, Wout, bout):
    raise NotImplementedError("write your pallas kernel here")


# speedup vs baseline: 23.4198x; 23.4198x over previous
import functools

import jax
import jax.numpy as jnp
from jax import lax
from jax.experimental import pallas as pl
from jax.experimental.pallas import tpu as pltpu
from jax.experimental.pallas import tpu_sc as plsc

B, HW, D = 4, 1024, 768
L = 4096
H, Dh = 8, 64
KN = 32
INNER = H * Dh
BHW = B * HW
BL = B * L
NW = 32
QW = BHW // NW
NV = INNER // 16


def _mm(a, b, bias=None, bm=512):
    M, K = a.shape
    _, N = b.shape
    in_specs = [pl.BlockSpec((bm, K), lambda i: (i, 0)),
                pl.BlockSpec((K, N), lambda i: (0, 0))]
    args = [a, b]
    has_bias = bias is not None
    if has_bias:
        in_specs.append(pl.BlockSpec((1, N), lambda i: (0, 0)))
        args.append(bias.reshape(1, N))

    def body(*refs):
        a_ref, b_ref = refs[0], refs[1]
        o_ref = refs[-1]
        acc = lax.dot_general(a_ref[...], b_ref[...], (((1,), (0,)), ((), ())),
                              preferred_element_type=jnp.float32,
                              precision=lax.Precision.HIGHEST)
        if has_bias:
            acc = acc + refs[2][...]
        o_ref[...] = acc

    return pl.pallas_call(
        body,
        grid=(M // bm,),
        in_specs=in_specs,
        out_specs=pl.BlockSpec((bm, N), lambda i: (i, 0)),
        out_shape=jax.ShapeDtypeStruct((M, N), jnp.float32),
    )(*args)


def _rot8(v):
    idx = lax.iota(jnp.int32, 16) ^ 8
    dnums = lax.GatherDimensionNumbers(
        offset_dims=(), collapsed_slice_dims=(0,), start_index_map=(0,))
    return lax.gather(v, idx[:, None], dnums, (1,),
                      mode=lax.GatherScatterMode.PROMISE_IN_BOUNDS)


def _sc_attn(q, k, v, idx, bias):
    mesh = plsc.VectorSubcoreMesh(core_axis_name="c", subcore_axis_name="s")

    @functools.partial(
        pl.kernel,
        out_type=jax.ShapeDtypeStruct((BHW, INNER), jnp.float32),
        mesh=mesh,
        scratch_types=[
            pltpu.VMEM((INNER,), jnp.float32),
            pltpu.VMEM((QW, KN), jnp.int32),
            pltpu.VMEM((QW, KN), jnp.float32),
            pltpu.VMEM((KN, INNER), jnp.float32),
            pltpu.VMEM((KN, INNER), jnp.float32),
            pltpu.VMEM((KN, 16), jnp.float32),
            pltpu.VMEM((INNER,), jnp.float32),
        ],
    )
    def body(q_hbm, k_hbm, v_hbm, idx_hbm, bias_hbm, o_hbm,
             qv, idxv, biasv, kg, vg, simv, outv):
        wid = lax.axis_index("s") * 2 + lax.axis_index("c")
        base = wid * QW
        pltpu.sync_copy(idx_hbm.at[pl.ds(base, QW)], idxv)
        pltpu.sync_copy(bias_hbm.at[pl.ds(base, QW)], biasv)

        @pl.loop(0, QW)
        def _(qi):
            pltpu.sync_copy(q_hbm.at[base + qi], qv)
            pltpu.sync_copy(k_hbm.at[idxv.at[qi]], kg)
            pltpu.sync_copy(v_hbm.at[idxv.at[qi]], vg)
            qvecs = [qv[pl.ds(16 * j, 16)] for j in range(NV)]
            bvecs = [biasv[qi, pl.ds(16 * j, 16)] for j in range(KN // 16)]
            for kk in range(KN):
                p = qvecs[0] * kg[kk, pl.ds(0, 16)]
                for j in range(1, NV):
                    p = p + qvecs[j] * kg[kk, pl.ds(16 * j, 16)]
                simv[kk, :] = p + _rot8(p) + bvecs[kk // 16][kk % 16]
            m = simv[0, :]
            for kk in range(1, KN):
                m = jnp.maximum(m, simv[kk, :])
            den = None
            for kk in range(KN):
                e = jnp.exp(simv[kk, :] - m)
                simv[kk, :] = e
                den = e if den is None else den + e
            inv = 1.0 / den
            accs = None
            for kk in range(KN):
                w = simv[kk, :]
                term = [w * vg[kk, pl.ds(16 * j, 16)] for j in range(NV)]
                accs = term if accs is None else [a + t for a, t in zip(accs, term)]
            for j in range(NV):
                outv[pl.ds(16 * j, 16)] = accs[j] * inv
            pltpu.sync_copy(outv, o_hbm.at[base + qi])

    return body(q, k, v, idx, bias)


def kernel(x, context, attn_indices, bias, Wq, Wkv, Wout, bout):
    scale = Dh ** (-0.5)
    Wq_p = (Wq * scale).reshape(D, H, Dh).transpose(0, 2, 1).reshape(D, INNER)
    Wk_p = Wkv[:, :INNER].reshape(D, H, Dh).transpose(0, 2, 1).reshape(D, INNER)
    Wv_p = Wkv[:, INNER:].reshape(D, H, Dh).transpose(0, 2, 1).reshape(D, INNER)
    Wout_p = Wout.reshape(H, Dh, D).transpose(1, 0, 2).reshape(INNER, D)

    qp = _mm(x.reshape(BHW, D), Wq_p)
    kp = _mm(context.reshape(BL, D), Wk_p)
    vp = _mm(context.reshape(BL, D), Wv_p)

    idx = (attn_indices.astype(jnp.int32)
           + (jnp.arange(B, dtype=jnp.int32) * L)[:, None, None]).reshape(BHW, KN)
    attn = _sc_attn(qp, kp, vp, idx, bias.reshape(BHW, KN).astype(jnp.float32))

    out = _mm(attn, Wout_p, bias=bout)
    return out.reshape(B, HW, D)


# speedup vs baseline: 36.6127x; 1.5633x over previous
import dataclasses
import functools

import jax
import jax.numpy as jnp
import numpy as np
from jax import lax
from jax.experimental import pallas as pl
from jax.experimental.pallas import tpu as pltpu
from jax.experimental.pallas import tpu_sc as plsc

B, HW, D = 4, 1024, 768
L = 4096
H, Dh = 8, 64
KN = 32
INNER = H * Dh
BHW = B * HW
BL = B * L
NW = 32
QW = BHW // NW
NV = INNER // 16
QC = 16

_cp = np.arange(INNER)
_PERM_F32 = np.asarray((_cp % 8) * Dh + _cp // 8)
_m = np.arange(INNER)
_cp2 = 32 * (_m // 32) + (_m % 2) * 16 + (_m % 32) // 2
_PERM_BF16 = np.asarray((_cp2 % 8) * Dh + _cp2 // 8)


def _mm(a, b, bias=None, bm=512, out_dtype=jnp.float32,
        precision=lax.Precision.DEFAULT):
    M, K = a.shape
    _, N = b.shape
    in_specs = [pl.BlockSpec((bm, K), lambda i: (i, 0)),
                pl.BlockSpec((K, N), lambda i: (0, 0))]
    args = [a, b]
    has_bias = bias is not None
    if has_bias:
        in_specs.append(pl.BlockSpec((1, N), lambda i: (0, 0)))
        args.append(bias.reshape(1, N))

    def body(*refs):
        a_ref, b_ref = refs[0], refs[1]
        o_ref = refs[-1]
        acc = lax.dot_general(a_ref[...], b_ref[...], (((1,), (0,)), ((), ())),
                              preferred_element_type=jnp.float32,
                              precision=precision)
        if has_bias:
            acc = acc + refs[2][...]
        o_ref[...] = acc.astype(out_dtype)

    return pl.pallas_call(
        body,
        grid=(M // bm,),
        in_specs=in_specs,
        out_specs=pl.BlockSpec((bm, N), lambda i: (i, 0)),
        out_shape=jax.ShapeDtypeStruct((M, N), out_dtype),
    )(*args)


def _rot8(v):
    idx = lax.iota(jnp.int32, 16) ^ 8
    dnums = lax.GatherDimensionNumbers(
        offset_dims=(), collapsed_slice_dims=(0,), start_index_map=(0,))
    return lax.gather(v, idx[:, None], dnums, (1,),
                      mode=lax.GatherScatterMode.PROMISE_IN_BOUNDS)


def _unpack_bf16(xi):
    a = plsc.bitcast(xi << 16, jnp.float32)
    b = plsc.bitcast(xi & jnp.int32(-65536), jnp.float32)
    return a, b


def _sc_attn(q, kv, idx, bias):
    mesh = plsc.VectorSubcoreMesh(core_axis_name="c", subcore_axis_name="s")
    cp = pltpu.CompilerParams()
    if "needs_layout_passes" in pltpu.CompilerParams.__dataclass_fields__:
        cp = dataclasses.replace(cp, needs_layout_passes=False)

    @functools.partial(
        pl.kernel,
        out_type=jax.ShapeDtypeStruct((BHW, INNER), jnp.float32),
        mesh=mesh,
        compiler_params=cp,
        scratch_types=[
            pltpu.VMEM((QC, INNER), jnp.float32),
            pltpu.VMEM((QW, KN), jnp.int32),
            pltpu.VMEM((QW, KN), jnp.float32),
            pltpu.VMEM((KN, INNER), jnp.int32),
            pltpu.VMEM((KN, 16), jnp.float32),
            pltpu.VMEM((QC, INNER), jnp.float32),
        ],
    )
    def body(q_hbm, kv_hbm, idx_hbm, bias_hbm, o_hbm,
             qv, idxv, biasv, kvg, simv, outv):
        wid = lax.axis_index("s") * 2 + lax.axis_index("c")
        base = wid * QW
        pltpu.sync_copy(idx_hbm.at[pl.ds(base, QW)], idxv)
        pltpu.sync_copy(bias_hbm.at[pl.ds(base, QW)], biasv)

        @pl.loop(0, QW, step=QC)
        def _(qc):
            pltpu.sync_copy(q_hbm.at[pl.ds(base + qc, QC)], qv)

            @pl.loop(0, QC)
            def _(t):
                qi = qc + t
                pltpu.sync_copy(kv_hbm.at[idxv.at[qi]], kvg)
                qvecs = [qv[t, pl.ds(16 * j, 16)] for j in range(NV)]
                bvecs = [biasv[qi, pl.ds(16 * j, 16)] for j in range(KN // 16)]
                for kk in range(KN):
                    p = None
                    for j in range(NV // 2):
                        ka, kb = _unpack_bf16(kvg[kk, pl.ds(16 * j, 16)])
                        term = qvecs[2 * j] * ka + qvecs[2 * j + 1] * kb
                        p = term if p is None else p + term
                    simv[kk, :] = p + _rot8(p) + bvecs[kk // 16][kk % 16]
                m = simv[0, :]
                for kk in range(1, KN):
                    m = jnp.maximum(m, simv[kk, :])
                den = None
                for kk in range(KN):
                    e = jnp.exp(simv[kk, :] - m)
                    simv[kk, :] = e
                    den = e if den is None else den + e
                inv = 1.0 / den
                accs = None
                for kk in range(KN):
                    w = simv[kk, :]
                    term = []
                    for j in range(NV // 2):
                        va, vb = _unpack_bf16(kvg[kk, pl.ds(INNER // 2 + 16 * j, 16)])
                        term += [w * va, w * vb]
                    accs = term if accs is None else [a + v for a, v in zip(accs, term)]
                for j in range(NV):
                    outv[t, pl.ds(16 * j, 16)] = accs[j] * inv

            pltpu.sync_copy(outv, o_hbm.at[pl.ds(base + qc, QC)])

    return body(q, kv, idx, bias)


def kernel(x, context, attn_indices, bias, Wq, Wkv, Wout, bout):
    scale = Dh ** (-0.5)
    Wq_p = (Wq * scale)[:, _PERM_F32]
    Wk_p = Wkv[:, :INNER][:, _PERM_BF16]
    Wv_p = Wkv[:, INNER:][:, _PERM_BF16]
    Wkv_p = jnp.concatenate([Wk_p, Wv_p], axis=1)
    Wout_p = Wout[_PERM_F32, :]

    qp = _mm(x.reshape(BHW, D), Wq_p)
    kvp = _mm(context.reshape(BL, D), Wkv_p, out_dtype=jnp.bfloat16)
    kvp = lax.bitcast_convert_type(kvp.reshape(BL, INNER, 2), jnp.int32)

    idx = (attn_indices.astype(jnp.int32)
           + (jnp.arange(B, dtype=jnp.int32) * L)[:, None, None]).reshape(BHW, KN)
    attn = _sc_attn(qp, kvp, idx, bias.reshape(BHW, KN).astype(jnp.float32))

    out = _mm(attn, Wout_p, bias=bout)
    return out.reshape(B, HW, D)


# speedup vs baseline: 39.4429x; 1.0773x over previous
import dataclasses
import functools

import jax
import jax.numpy as jnp
import numpy as np
from jax import lax
from jax.experimental import pallas as pl
from jax.experimental.pallas import tpu as pltpu
from jax.experimental.pallas import tpu_sc as plsc

B, HW, D = 4, 1024, 768
L = 4096
H, Dh = 8, 64
KN = 32
INNER = H * Dh
BHW = B * HW
BL = B * L
NW = 32
QW = BHW // NW
NV = INNER // 16
QC = 16

_cp = np.arange(INNER)
_PERM_F32 = np.asarray((_cp % 8) * Dh + _cp // 8)
_m = np.arange(INNER)
_cp2 = 32 * (_m // 32) + (_m % 2) * 16 + (_m % 32) // 2
_PERM_BF16 = np.asarray((_cp2 % 8) * Dh + _cp2 // 8)


def _mm(a, b, bias=None, bm=512, out_dtype=jnp.float32,
        precision=lax.Precision.DEFAULT):
    M, K = a.shape
    _, N = b.shape
    in_specs = [pl.BlockSpec((bm, K), lambda i: (i, 0)),
                pl.BlockSpec((K, N), lambda i: (0, 0))]
    args = [a, b]
    has_bias = bias is not None
    if has_bias:
        in_specs.append(pl.BlockSpec((1, N), lambda i: (0, 0)))
        args.append(bias.reshape(1, N))

    def body(*refs):
        a_ref, b_ref = refs[0], refs[1]
        o_ref = refs[-1]
        acc = lax.dot_general(a_ref[...], b_ref[...], (((1,), (0,)), ((), ())),
                              preferred_element_type=jnp.float32,
                              precision=precision)
        if has_bias:
            acc = acc + refs[2][...]
        o_ref[...] = acc.astype(out_dtype)

    return pl.pallas_call(
        body,
        grid=(M // bm,),
        in_specs=in_specs,
        out_specs=pl.BlockSpec((bm, N), lambda i: (i, 0)),
        out_shape=jax.ShapeDtypeStruct((M, N), out_dtype),
    )(*args)


def _rot8(v):
    idx = lax.iota(jnp.int32, 16) ^ 8
    dnums = lax.GatherDimensionNumbers(
        offset_dims=(), collapsed_slice_dims=(0,), start_index_map=(0,))
    return lax.gather(v, idx[:, None], dnums, (1,),
                      mode=lax.GatherScatterMode.PROMISE_IN_BOUNDS)


def _unpack_bf16(xi):
    a = plsc.bitcast(xi << 16, jnp.float32)
    b = plsc.bitcast(xi & jnp.int32(-65536), jnp.float32)
    return a, b


def _sc_attn(q, kv, idx, bias):
    mesh = plsc.VectorSubcoreMesh(core_axis_name="c", subcore_axis_name="s")
    cp = pltpu.CompilerParams()
    if "needs_layout_passes" in pltpu.CompilerParams.__dataclass_fields__:
        cp = dataclasses.replace(cp, needs_layout_passes=False)

    @functools.partial(
        pl.kernel,
        out_type=jax.ShapeDtypeStruct((BHW, INNER), jnp.float32),
        mesh=mesh,
        compiler_params=cp,
        scratch_types=[
            pltpu.VMEM((QC, INNER), jnp.float32),
            pltpu.VMEM((QW, KN), jnp.int32),
            pltpu.VMEM((QW, KN), jnp.float32),
            pltpu.VMEM((KN, INNER), jnp.int32),
            pltpu.VMEM((KN, INNER), jnp.int32),
            pltpu.VMEM((KN, 16), jnp.float32),
            pltpu.VMEM((QC, INNER), jnp.float32),
            pltpu.SemaphoreType.DMA,
            pltpu.SemaphoreType.DMA,
        ],
    )
    def body(q_hbm, kv_hbm, idx_hbm, bias_hbm, o_hbm,
             qv, idxv, biasv, kvg0, kvg1, simv, outv, sem0, sem1):
        wid = lax.axis_index("s") * 2 + lax.axis_index("c")
        base = wid * QW
        pltpu.sync_copy(idx_hbm.at[pl.ds(base, QW)], idxv)
        pltpu.sync_copy(bias_hbm.at[pl.ds(base, QW)], biasv)

        def start_gather(qi, buf, sem):
            pltpu.async_copy(kv_hbm.at[idxv.at[qi]], buf, sem)

        def wait_gather(buf, sem):
            pltpu.make_async_copy(kv_hbm.at[idxv.at[0]], buf, sem).wait()

        def compute(qi, t, kvg):
            qvecs = [qv[t, pl.ds(16 * j, 16)] for j in range(NV)]
            bvecs = [biasv[qi, pl.ds(16 * j, 16)] for j in range(KN // 16)]
            for kk in range(KN):
                p = None
                for j in range(NV // 2):
                    ka, kb = _unpack_bf16(kvg[kk, pl.ds(16 * j, 16)])
                    term = qvecs[2 * j] * ka + qvecs[2 * j + 1] * kb
                    p = term if p is None else p + term
                simv[kk, :] = p + _rot8(p) + bvecs[kk // 16][kk % 16]
            m = simv[0, :]
            for kk in range(1, KN):
                m = jnp.maximum(m, simv[kk, :])
            den = None
            for kk in range(KN):
                e = jnp.exp(simv[kk, :] - m)
                simv[kk, :] = e
                den = e if den is None else den + e
            inv = 1.0 / den
            accs = None
            for kk in range(KN):
                w = simv[kk, :]
                term = []
                for j in range(NV // 2):
                    va, vb = _unpack_bf16(kvg[kk, pl.ds(INNER // 2 + 16 * j, 16)])
                    term += [w * va, w * vb]
                accs = term if accs is None else [a + v for a, v in zip(accs, term)]
            for j in range(NV):
                outv[t, pl.ds(16 * j, 16)] = accs[j] * inv

        start_gather(0, kvg0, sem0)

        @pl.loop(0, QW, step=2)
        def _(qi):
            t = lax.rem(qi, QC)
            qc = pl.multiple_of(qi - t, QC)

            @pl.when(t == 0)
            def _():
                pltpu.sync_copy(q_hbm.at[pl.ds(base + qc, QC)], qv)

            start_gather(qi + 1, kvg1, sem1)
            wait_gather(kvg0, sem0)
            compute(qi, t, kvg0)

            @pl.when(qi + 2 < QW)
            def _():
                start_gather(qi + 2, kvg0, sem0)

            wait_gather(kvg1, sem1)
            compute(qi + 1, t + 1, kvg1)

            @pl.when(t + 2 == QC)
            def _():
                pltpu.sync_copy(outv, o_hbm.at[pl.ds(base + qc, QC)])

    return body(q, kv, idx, bias)


def kernel(x, context, attn_indices, bias, Wq, Wkv, Wout, bout):
    scale = Dh ** (-0.5)
    Wq_p = (Wq * scale)[:, _PERM_F32]
    Wk_p = Wkv[:, :INNER][:, _PERM_BF16]
    Wv_p = Wkv[:, INNER:][:, _PERM_BF16]
    Wkv_p = jnp.concatenate([Wk_p, Wv_p], axis=1)
    Wout_p = Wout[_PERM_F32, :]

    qp = _mm(x.reshape(BHW, D), Wq_p)
    kvp = _mm(context.reshape(BL, D), Wkv_p, out_dtype=jnp.bfloat16)
    kvp = lax.bitcast_convert_type(kvp.reshape(BL, INNER, 2), jnp.int32)

    idx = (attn_indices.astype(jnp.int32)
           + (jnp.arange(B, dtype=jnp.int32) * L)[:, None, None]).reshape(BHW, KN)
    attn = _sc_attn(qp, kvp, idx, bias.reshape(BHW, KN).astype(jnp.float32))

    out = _mm(attn, Wout_p, bias=bout)
    return out.reshape(B, HW, D)


# speedup vs baseline: 66.7496x; 1.6923x over previous
import dataclasses
import functools

import jax
import jax.numpy as jnp
import numpy as np
from jax import lax
from jax.experimental import pallas as pl
from jax.experimental.pallas import tpu as pltpu
from jax.experimental.pallas import tpu_sc as plsc

B, HW, D = 4, 1024, 768
L = 4096
H, Dh = 8, 64
KN = 32
INNER = H * Dh
NW = 32
QW = HW // NW
NV = INNER // 16
QC = 16

_cp = np.arange(INNER)
_PERM_F32 = np.asarray((_cp % 8) * Dh + _cp // 8)


def _kv_perm():
    w = np.arange(2 * INNER)
    is_hi = w >= 512
    base = np.where(is_hi, w - 512, w)
    is_v = base >= 256
    wv = np.where(is_v, base - 256, base)
    cp = 32 * (wv // 16) + (wv % 16) + 16 * is_hi
    return np.asarray((cp % 8) * Dh + cp // 8 + INNER * is_v)


_PERM_KV = _kv_perm()


def _mm(a, b, bias=None, bm=512, pack_kv=False):
    M, K = a.shape
    _, N = b.shape
    in_specs = [pl.BlockSpec((bm, K), lambda i: (i, 0)),
                pl.BlockSpec((K, N), lambda i: (0, 0))]
    args = [a, b]
    has_bias = bias is not None
    if has_bias:
        in_specs.append(pl.BlockSpec((1, N), lambda i: (0, 0)))
        args.append(bias.reshape(1, N))

    def body(*refs):
        a_ref, b_ref = refs[0], refs[1]
        o_ref = refs[-1]
        acc = lax.dot_general(a_ref[...], b_ref[...], (((1,), (0,)), ((), ())),
                              preferred_element_type=jnp.float32,
                              precision=lax.Precision.DEFAULT)
        if has_bias:
            acc = acc + refs[2][...]
        if pack_kv:
            half = N // 2
            lo = lax.bitcast_convert_type(
                acc[:, :half].astype(jnp.bfloat16), jnp.uint16).astype(jnp.uint32)
            hi = lax.bitcast_convert_type(
                acc[:, half:].astype(jnp.bfloat16), jnp.uint16).astype(jnp.uint32)
            o_ref[...] = lax.bitcast_convert_type(lo | (hi << 16), jnp.int32)
        else:
            o_ref[...] = acc

    out_n = N // 2 if pack_kv else N
    out_dtype = jnp.int32 if pack_kv else jnp.float32
    return pl.pallas_call(
        body,
        grid=(M // bm,),
        in_specs=in_specs,
        out_specs=pl.BlockSpec((bm, out_n), lambda i: (i, 0)),
        out_shape=jax.ShapeDtypeStruct((M, out_n), out_dtype),
    )(*args)


def _rot8(v):
    idx = lax.iota(jnp.int32, 16) ^ 8
    dnums = lax.GatherDimensionNumbers(
        offset_dims=(), collapsed_slice_dims=(0,), start_index_map=(0,))
    return lax.gather(v, idx[:, None], dnums, (1,),
                      mode=lax.GatherScatterMode.PROMISE_IN_BOUNDS)


def _unpack_bf16(xi):
    a = plsc.bitcast(xi << 16, jnp.float32)
    b = plsc.bitcast(xi, jnp.float32)
    return a, b


def _sc_attn(q, kv, idx, bias):
    mesh = plsc.VectorSubcoreMesh(core_axis_name="c", subcore_axis_name="s")
    cp = pltpu.CompilerParams()
    if "needs_layout_passes" in pltpu.CompilerParams.__dataclass_fields__:
        cp = dataclasses.replace(cp, needs_layout_passes=False)

    @functools.partial(
        pl.kernel,
        out_type=jax.ShapeDtypeStruct((HW, INNER), jnp.float32),
        mesh=mesh,
        compiler_params=cp,
        scratch_types=[
            pltpu.VMEM((QC, INNER), jnp.float32),
            pltpu.VMEM((QW, KN), jnp.int32),
            pltpu.VMEM((QW, KN), jnp.float32),
            pltpu.VMEM((KN, INNER), jnp.int32),
            pltpu.VMEM((KN, INNER), jnp.int32),
            pltpu.VMEM((KN, 16), jnp.float32),
            pltpu.VMEM((QC, INNER), jnp.float32),
            pltpu.SemaphoreType.DMA,
            pltpu.SemaphoreType.DMA,
        ],
    )
    def body(q_hbm, kv_hbm, idx_hbm, bias_hbm, o_hbm,
             qv, idxv, biasv, kvg0, kvg1, simv, outv, sem0, sem1):
        wid = lax.axis_index("s") * 2 + lax.axis_index("c")
        base = wid * QW
        pltpu.sync_copy(idx_hbm.at[pl.ds(base, QW)], idxv)
        pltpu.sync_copy(bias_hbm.at[pl.ds(base, QW)], biasv)

        def start_gather(qi, buf, sem):
            pltpu.async_copy(kv_hbm.at[idxv.at[qi]], buf, sem)

        def wait_gather(buf, sem):
            pltpu.make_async_copy(kv_hbm.at[idxv.at[0]], buf, sem).wait()

        def compute(qi, t, kvg):
            qvecs = [qv[t, pl.ds(16 * j, 16)] for j in range(NV)]
            bvecs = [biasv[qi, pl.ds(16 * j, 16)] for j in range(KN // 16)]
            den = None
            for kk in range(KN):
                p = None
                for j in range(NV // 2):
                    ka, kb = _unpack_bf16(kvg[kk, pl.ds(16 * j, 16)])
                    term = qvecs[2 * j] * ka + qvecs[2 * j + 1] * kb
                    p = term if p is None else p + term
                e = jnp.exp(p + _rot8(p) + bvecs[kk // 16][kk % 16])
                simv[kk, :] = e
                den = e if den is None else den + e
            inv = 1.0 / den
            accs = None
            for kk in range(KN):
                w = simv[kk, :]
                term = []
                for j in range(NV // 2):
                    va, vb = _unpack_bf16(kvg[kk, pl.ds(INNER // 2 + 16 * j, 16)])
                    term += [w * va, w * vb]
                accs = term if accs is None else [a + v for a, v in zip(accs, term)]
            for j in range(NV):
                outv[t, pl.ds(16 * j, 16)] = accs[j] * inv

        start_gather(0, kvg0, sem0)

        @pl.loop(0, QW, step=2)
        def _(qi):
            t = lax.rem(qi, QC)
            qc = pl.multiple_of(qi - t, QC)

            @pl.when(t == 0)
            def _():
                pltpu.sync_copy(q_hbm.at[pl.ds(base + qc, QC)], qv)

            start_gather(qi + 1, kvg1, sem1)
            wait_gather(kvg0, sem0)
            compute(qi, t, kvg0)

            @pl.when(qi + 2 < QW)
            def _():
                start_gather(qi + 2, kvg0, sem0)

            wait_gather(kvg1, sem1)
            compute(qi + 1, t + 1, kvg1)

            @pl.when(t + 2 == QC)
            def _():
                pltpu.sync_copy(outv, o_hbm.at[pl.ds(base + qc, QC)])

    return body(q, kv, idx, bias)


def kernel(x, context, attn_indices, bias, Wq, Wkv, Wout, bout):
    scale = Dh ** (-0.5)
    Wq_p = (Wq * scale)[:, _PERM_F32]
    Wkv_p = Wkv[:, _PERM_KV]
    Wout_p = Wout[_PERM_F32, :]
    idx = attn_indices.astype(jnp.int32)
    bias = bias.astype(jnp.float32)

    outs = []
    for b in range(B):
        qp = _mm(x[b], Wq_p)
        kvp = _mm(context[b], Wkv_p, pack_kv=True)
        attn = _sc_attn(qp, kvp, idx[b], bias[b])
        outs.append(_mm(attn, Wout_p, bias=bout))
    return jnp.stack(outs)
